# Initial kernel scaffold; baseline (speedup 1.0000x reference)
import jax
import jax.numpy as jnp
from jax.experimental import pallas as pl


def kernel(x, edge_index, W1l, W1r, b1, W2l, W2r, b2, W3l, W3r, b3):
    raise NotImplementedError("write your pallas kernel here")


# speedup vs baseline: 18.1806x; 18.1806x over previous
import functools

import jax
import jax.numpy as jnp
from jax import lax
from jax.experimental import pallas as pl
from jax.experimental.pallas import tpu as pltpu
from jax.experimental.pallas import tpu_sc as plsc

N = 100000
E = 6400000
C = 128
NCHUNK = E // C
NC = 2
NS = 16
NW = NC * NS
RPT = 6256
RPT_LAST = N - (NS - 1) * RPT
HIGH = lax.Precision.HIGHEST

_mesh = functools.partial(
    plsc.VectorSubcoreMesh, core_axis_name="c", subcore_axis_name="s")
_SC_PARAMS = pltpu.CompilerParams(use_tc_tiling_on_sc=False)


def _per_tile(sid, op):
    @pl.when(sid < NS - 1)
    def _():
        op(sid * RPT, RPT)

    @pl.when(sid == NS - 1)
    def _():
        op((NS - 1) * RPT, RPT_LAST)


def _edge_loop(cid, sid, body):
    wid = sid * NC + cid
    nt = lax.div(NCHUNK - wid + NW - 1, NW)
    lax.fori_loop(0, nt, lambda t, carry: body(t * NW + wid), 0)


@functools.partial(
    pl.kernel,
    mesh=_mesh(),
    out_type=(
        jax.ShapeDtypeStruct((NC, N, 8), jnp.float32),
        jax.ShapeDtypeStruct((NC * N,), jnp.float32),
        jax.ShapeDtypeStruct((NC * N,), jnp.float32),
    ),
    scratch_types=[
        pltpu.VMEM((C,), jnp.int32),
        pltpu.VMEM((C,), jnp.int32),
        pltpu.VMEM((C, 8), jnp.float32),
        pltpu.VMEM((C,), jnp.float32),
        pltpu.VMEM((C,), jnp.float32),
        pltpu.VMEM((RPT,), jnp.float32),
        pltpu.VMEM_SHARED((N, 8), jnp.float32),
        pltpu.VMEM_SHARED((N,), jnp.float32),
        pltpu.VMEM_SHARED((N,), jnp.float32),
        pltpu.SemaphoreType.DMA,
    ],
    compiler_params=_SC_PARAMS,
)
def _agg_pass_a(src_h, dst_h, tbl_h, z8_h, z1_h, acc_out, cnt0_out, cnt1_out,
                src_v, dst_v, rows_v, cnt0_v, cnt1_v, bounce_v, acc_s,
                cnt0_s, cnt1_s, sem):
    cid = lax.axis_index("c")
    sid = lax.axis_index("s")

    def _init(start, size):
        sl = pl.ds(start, size)
        bsl = pl.ds(0, size)
        pltpu.sync_copy(z8_h.at[sl], acc_s.at[sl])
        pltpu.sync_copy(z1_h.at[sl], bounce_v.at[bsl])
        pltpu.sync_copy(bounce_v.at[bsl], cnt0_s.at[sl])
        pltpu.sync_copy(bounce_v.at[bsl], cnt1_s.at[sl])

    _per_tile(sid, _init)
    plsc.subcore_barrier()

    def body(chunk):
        base = chunk * C
        pltpu.sync_copy(src_h.at[pl.ds(base, C)], src_v)
        pltpu.sync_copy(dst_h.at[pl.ds(base, C)], dst_v)
        pltpu.async_copy(tbl_h.at[src_v], rows_v, sem).wait()
        pltpu.sync_copy(rows_v, acc_s.at[dst_v], add=True)
        one = jnp.ones((16,), jnp.float32)
        zero = jnp.zeros((16,), jnp.float32)
        for kk in range(C // 16):
            d16 = dst_v[pl.ds(kk * 16, 16)]
            cnt0_v[pl.ds(kk * 16, 16)] = jnp.where(d16 == 0, one, zero)
            cnt1_v[pl.ds(kk * 16, 16)] = jnp.where(d16 == 1, one, zero)
        pltpu.sync_copy(cnt0_v, cnt0_s.at[src_v], add=True)
        pltpu.sync_copy(cnt1_v, cnt1_s.at[src_v], add=True)
        return 0

    _edge_loop(cid, sid, body)
    plsc.subcore_barrier()

    def _writeout(start, size):
        sl = pl.ds(start, size)
        bsl = pl.ds(0, size)
        flat = pl.ds(cid * N + start, size)
        pltpu.sync_copy(acc_s.at[sl], acc_out.at[cid, sl])
        pltpu.sync_copy(cnt0_s.at[sl], bounce_v.at[bsl])
        pltpu.sync_copy(bounce_v.at[bsl], cnt0_out.at[flat])
        pltpu.sync_copy(cnt1_s.at[sl], bounce_v.at[bsl])
        pltpu.sync_copy(bounce_v.at[bsl], cnt1_out.at[flat])

    _per_tile(sid, _writeout)


@functools.partial(
    pl.kernel,
    mesh=_mesh(),
    out_type=jax.ShapeDtypeStruct((NC, N, 16), jnp.float32),
    scratch_types=[
        pltpu.VMEM((C,), jnp.int32),
        pltpu.VMEM((C,), jnp.int32),
        pltpu.VMEM((C, 16), jnp.float32),
        pltpu.VMEM_SHARED((N, 16), jnp.float32),
        pltpu.SemaphoreType.DMA,
    ],
    compiler_params=_SC_PARAMS,
)
def _agg_pass_b(src_h, dst_h, tbl_h, z16_h, acc_out,
                src_v, dst_v, rows_v, acc_s, sem):
    cid = lax.axis_index("c")
    sid = lax.axis_index("s")
    _per_tile(sid, lambda start, size: pltpu.sync_copy(
        z16_h.at[pl.ds(start, size)], acc_s.at[pl.ds(start, size)]))
    plsc.subcore_barrier()

    def body(chunk):
        base = chunk * C
        pltpu.sync_copy(src_h.at[pl.ds(base, C)], src_v)
        pltpu.sync_copy(dst_h.at[pl.ds(base, C)], dst_v)
        pltpu.async_copy(tbl_h.at[src_v], rows_v, sem).wait()
        pltpu.sync_copy(rows_v, acc_s.at[dst_v], add=True)
        return 0

    _edge_loop(cid, sid, body)
    plsc.subcore_barrier()
    _per_tile(sid, lambda start, size: pltpu.sync_copy(
        acc_s.at[pl.ds(start, size)], acc_out.at[cid, pl.ds(start, size)]))


BD = 2000
G = N // BD


def _dense1_body(acc_ref, cnt0_ref, cnt1_ref, x_ref, wl_ref, wr_ref, b_ref,
                 h1p_ref, cnt_ref):
    acc = acc_ref[...]
    agg = acc[0, :, :6] + acc[1, :, :6]
    deg = acc[0, :, 6] + acc[1, :, 6]
    dinv = 1.0 / jnp.clip(deg, 1.0, None)
    mean = agg * dinv[:, None]
    xb = x_ref[...][:, :6]
    h1 = lax.dot_general(mean, wl_ref[...], (((1,), (1,)), ((), ())),
                         precision=HIGH, preferred_element_type=jnp.float32)
    h1 = h1 + lax.dot_general(xb, wr_ref[...], (((1,), (1,)), ((), ())),
                              precision=HIGH,
                              preferred_element_type=jnp.float32)
    h1 = jnp.maximum(h1 + b_ref[...], 0.0)
    h1p_ref[...] = jnp.concatenate(
        [h1, dinv[:, None], jnp.zeros((BD, 3), jnp.float32)], axis=1)
    c0 = cnt0_ref[...]
    c1 = cnt1_ref[...]
    cnt_ref[...] = jnp.concatenate(
        [c0[0] + c0[1], c1[0] + c1[1]], axis=1)


def _dense2_body(acc_ref, h1p_ref, cnt_ref, w2l_ref, w2r_ref, b2_ref,
                 w3l_ref, w3r_ref, b3_ref, out_ref, agg3_s, h2f_s, dinv_s):
    i = pl.program_id(0)
    acc = acc_ref[...]
    h1p = h1p_ref[...]
    dinv = h1p[:, 12]
    mean2 = (acc[0, :, :12] + acc[1, :, :12]) * dinv[:, None]
    h2 = lax.dot_general(mean2, w2l_ref[...], (((1,), (1,)), ((), ())),
                         precision=HIGH, preferred_element_type=jnp.float32)
    h2 = h2 + lax.dot_general(h1p[:, :12], w2r_ref[...],
                              (((1,), (1,)), ((), ())),
                              precision=HIGH,
                              preferred_element_type=jnp.float32)
    h2 = jnp.maximum(h2 + b2_ref[...], 0.0)
    part = lax.dot_general(cnt_ref[...], h2, (((0,), (0,)), ((), ())),
                           precision=HIGH, preferred_element_type=jnp.float32)

    @pl.when(i == 0)
    def _():
        agg3_s[...] = part
        h2f_s[...] = h2[0:2, :]
        dinv_s[...] = dinv[0:2][:, None]

    @pl.when(i > 0)
    def _():
        agg3_s[...] = agg3_s[...] + part

    @pl.when(i == G - 1)
    def _():
        mean3 = agg3_s[...] * dinv_s[...]
        o = lax.dot_general(mean3, w3l_ref[...], (((1,), (1,)), ((), ())),
                            precision=HIGH, preferred_element_type=jnp.float32)
        o = o + lax.dot_general(h2f_s[...], w3r_ref[...],
                                (((1,), (1,)), ((), ())),
                                precision=HIGH,
                                preferred_element_type=jnp.float32)
        out_ref[...] = o + b3_ref[...]


_dense1 = pl.pallas_call(
    _dense1_body,
    grid=(G,),
    in_specs=[
        pl.BlockSpec((2, BD, 8), lambda i: (0, i, 0)),
        pl.BlockSpec((2, BD, 1), lambda i: (0, i, 0)),
        pl.BlockSpec((2, BD, 1), lambda i: (0, i, 0)),
        pl.BlockSpec((BD, 8), lambda i: (i, 0)),
        pl.BlockSpec((12, 6), lambda i: (0, 0)),
        pl.BlockSpec((12, 6), lambda i: (0, 0)),
        pl.BlockSpec((1, 12), lambda i: (0, 0)),
    ],
    out_specs=[
        pl.BlockSpec((BD, 16), lambda i: (i, 0)),
        pl.BlockSpec((BD, 2), lambda i: (i, 0)),
    ],
    out_shape=[
        jax.ShapeDtypeStruct((N, 16), jnp.float32),
        jax.ShapeDtypeStruct((N, 2), jnp.float32),
    ],
)

_dense2 = pl.pallas_call(
    _dense2_body,
    grid=(G,),
    in_specs=[
        pl.BlockSpec((2, BD, 16), lambda i: (0, i, 0)),
        pl.BlockSpec((BD, 16), lambda i: (i, 0)),
        pl.BlockSpec((BD, 2), lambda i: (i, 0)),
        pl.BlockSpec((24, 12), lambda i: (0, 0)),
        pl.BlockSpec((24, 12), lambda i: (0, 0)),
        pl.BlockSpec((1, 24), lambda i: (0, 0)),
        pl.BlockSpec((6, 24), lambda i: (0, 0)),
        pl.BlockSpec((6, 24), lambda i: (0, 0)),
        pl.BlockSpec((1, 6), lambda i: (0, 0)),
    ],
    out_specs=pl.BlockSpec((2, 6), lambda i: (0, 0)),
    out_shape=jax.ShapeDtypeStruct((2, 6), jnp.float32),
    scratch_shapes=[
        pltpu.VMEM((2, 24), jnp.float32),
        pltpu.VMEM((2, 24), jnp.float32),
        pltpu.VMEM((2, 1), jnp.float32),
    ],
)


def kernel(x, edge_index, W1l, W1r, b1, W2l, W2r, b2, W3l, W3r, b3):
    src = edge_index[0]
    dst = edge_index[1]
    xpad = jnp.concatenate(
        [x, jnp.ones((N, 1), jnp.float32), jnp.zeros((N, 1), jnp.float32)],
        axis=1)
    z8 = jnp.zeros((N, 8), jnp.float32)
    z1 = jnp.zeros((N,), jnp.float32)
    z16 = jnp.zeros((N, 16), jnp.float32)

    accA, cnt0A, cnt1A = _agg_pass_a(src, dst, xpad, z8, z1)
    h1pad, cnt = _dense1(accA, cnt0A.reshape(NC, N, 1),
                         cnt1A.reshape(NC, N, 1), xpad,
                         W1l, W1r, b1.reshape(1, 12))
    accB = _agg_pass_b(src, dst, h1pad, z16)
    out01 = _dense2(accB, h1pad, cnt, W2l, W2r, b2.reshape(1, 24),
                    W3l, W3r, b3.reshape(1, 6))
    return (out01[0], out01[1])
